# Initial kernel scaffold; baseline (speedup 1.0000x reference)
import functools

import jax
import jax.numpy as jnp
from jax import lax
from jax.experimental import pallas as pl
from jax.experimental.pallas import tpu as pltpu
from jax.experimental.pallas import tpu_sc as plsc

VOCAB = 1000000
D = 64
SEQ = 4096
N_NODES = 50000
SPAN = 8

NC = 2
NS = 16
LANES = 16

DCHUNKS = D // LANES
NGROUPS = (NC * NS) // DCHUNKS
NODES_PER_GROUP = N_NODES // NGROUPS
CHUNK = 125
NCHUNKS = NODES_PER_GROUP // CHUNK
IDX_ROWS = 32
IDX_COLS = SEQ // IDX_ROWS


def _body(tok_hbm, span_hbm, table4_hbm, out_hbm,
          tok_v, idx4_v, emb_v, span_v, out_v, span_s, sem):
    wid = lax.axis_index("s") * NC + lax.axis_index("c")
    cchunk = wid % DCHUNKS
    group = wid // DCHUNKS

    pltpu.sync_copy(tok_hbm, tok_v)

    iota = lax.iota(jnp.int32, LANES)
    for r in range(IDX_ROWS):
        for q in range(IDX_COLS // LANES):
            t = tok_v[pl.ds(r * IDX_COLS + q * LANES, LANES)]
            idx4_v[r, pl.ds(q * LANES, LANES)] = t * DCHUNKS + cchunk

    copies = []
    for r in range(IDX_ROWS):
        copies.append(pltpu.async_copy(
            table4_hbm.at[idx4_v.at[r]],
            emb_v.at[pl.ds(r * IDX_COLS, IDX_COLS)],
            sem))
    for cp in copies:
        cp.wait()

    def chunk_body(t, carry):
        node_base = group * NODES_PER_GROUP + t * CHUNK
        pltpu.sync_copy(span_hbm.at[pl.ds(node_base * SPAN, CHUNK * SPAN)],
                        span_v)
        pltpu.sync_copy(span_v, span_s)

        def node_body(k, carry2):
            s0 = span_s[k * SPAN]
            acc = plsc.load_gather(emb_v, [jnp.full((LANES,), s0), iota])
            for j in range(1, SPAN):
                sj = span_s[k * SPAN + j]
                acc = acc + plsc.load_gather(
                    emb_v, [jnp.full((LANES,), sj), iota])
            plsc.store_scatter(out_v, [jnp.full((LANES,), k), iota],
                               acc * jnp.float32(1.0 / SPAN))
            return carry2

        lax.fori_loop(0, CHUNK, node_body, 0)
        pltpu.sync_copy(out_v,
                        out_hbm.at[pl.ds(node_base, CHUNK),
                                   pl.ds(cchunk * LANES, LANES)])
        return carry

    lax.fori_loop(0, NCHUNKS, chunk_body, 0)


@jax.jit
def _graph_embed(tok, span, table4):
    mesh = plsc.VectorSubcoreMesh(core_axis_name="c", subcore_axis_name="s",
                                  num_cores=NC, num_subcores=NS)
    f = pl.kernel(
        _body,
        out_type=jax.ShapeDtypeStruct((N_NODES, D), jnp.float32),
        mesh=mesh,
        scratch_types=[
            pltpu.VMEM((SEQ,), jnp.int32),
            pltpu.VMEM((IDX_ROWS, IDX_COLS), jnp.int32),
            pltpu.VMEM((SEQ, LANES), jnp.float32),
            pltpu.VMEM((CHUNK * SPAN,), jnp.int32),
            pltpu.VMEM((CHUNK, LANES), jnp.float32),
            pltpu.SMEM((CHUNK * SPAN,), jnp.int32),
            pltpu.SemaphoreType.DMA,
        ],
    )
    return f(tok, span, table4)


def kernel(token_ids, node_span_idx, table):
    tok = token_ids.reshape(-1).astype(jnp.int32)
    span = node_span_idx.reshape(-1).astype(jnp.int32)
    table4 = table.reshape(-1, LANES)
    return _graph_embed(tok, span, table4)


# speedup vs baseline: 1.5307x; 1.5307x over previous
import functools

import jax
import jax.numpy as jnp
from jax import lax
from jax.experimental import pallas as pl
from jax.experimental.pallas import tpu as pltpu
from jax.experimental.pallas import tpu_sc as plsc

VOCAB = 1000000
D = 64
SEQ = 4096
N_NODES = 50000
SPAN = 8

NC = 2
NS = 16
LANES = 16

DCHUNKS = D // LANES
NGROUPS = (NC * NS) // DCHUNKS
NODES_PER_GROUP = N_NODES // NGROUPS
CHUNK = 250
NCHUNKS = NODES_PER_GROUP // CHUNK
IDX_ROWS = 32
IDX_COLS = SEQ // IDX_ROWS


def _body(tok_hbm, span_hbm, table4_hbm, out_hbm,
          tok_v, idx4_v, emb_v, span_v, out_v, sem):
    wid = lax.axis_index("s") * NC + lax.axis_index("c")
    cchunk = wid % DCHUNKS
    group = wid // DCHUNKS

    pltpu.sync_copy(tok_hbm, tok_v)

    iota = lax.iota(jnp.int32, LANES)
    for r in range(IDX_ROWS):
        for q in range(IDX_COLS // LANES):
            t = tok_v[pl.ds(r * IDX_COLS + q * LANES, LANES)]
            idx4_v[r, pl.ds(q * LANES, LANES)] = t * DCHUNKS + cchunk

    copies = []
    for r in range(IDX_ROWS):
        copies.append(pltpu.async_copy(
            table4_hbm.at[idx4_v.at[r]],
            emb_v.at[pl.ds(r * IDX_COLS, IDX_COLS)],
            sem))
    for cp in copies:
        cp.wait()

    def chunk_body(t, carry):
        node_base = group * NODES_PER_GROUP + t * CHUNK
        pltpu.sync_copy(span_hbm.at[pl.ds(node_base * SPAN, CHUNK * SPAN)],
                        span_v)

        def pair_body(k2, carry2):
            sv = span_v[pl.ds(k2 * 2 * SPAN, 2 * SPAN)]
            scale = jnp.float32(1.0 / SPAN)
            for half in range(2):
                acc = plsc.load_gather(
                    emb_v, [jnp.full((LANES,), sv[half * SPAN]), iota])
                for j in range(1, SPAN):
                    acc = acc + plsc.load_gather(
                        emb_v,
                        [jnp.full((LANES,), sv[half * SPAN + j]), iota])
                plsc.store_scatter(
                    out_v, [jnp.full((LANES,), k2 * 2 + half), iota],
                    acc * scale)
            return carry2

        lax.fori_loop(0, CHUNK // 2, pair_body, 0)
        pltpu.sync_copy(out_v,
                        out_hbm.at[pl.ds(node_base, CHUNK),
                                   pl.ds(cchunk * LANES, LANES)])
        return carry

    lax.fori_loop(0, NCHUNKS, chunk_body, 0)


@jax.jit
def _graph_embed(tok, span, table4):
    mesh = plsc.VectorSubcoreMesh(core_axis_name="c", subcore_axis_name="s",
                                  num_cores=NC, num_subcores=NS)
    f = pl.kernel(
        _body,
        out_type=jax.ShapeDtypeStruct((N_NODES, D), jnp.float32),
        mesh=mesh,
        scratch_types=[
            pltpu.VMEM((SEQ,), jnp.int32),
            pltpu.VMEM((IDX_ROWS, IDX_COLS), jnp.int32),
            pltpu.VMEM((SEQ, LANES), jnp.float32),
            pltpu.VMEM((CHUNK * SPAN,), jnp.int32),
            pltpu.VMEM((CHUNK, LANES), jnp.float32),
            pltpu.SemaphoreType.DMA,
        ],
        compiler_params=pltpu.CompilerParams(use_tc_tiling_on_sc=False,
                                             needs_layout_passes=False),
    )
    return f(tok, span, table4)


def kernel(token_ids, node_span_idx, table):
    tok = token_ids.reshape(-1).astype(jnp.int32)
    span = node_span_idx.reshape(-1).astype(jnp.int32)
    table4 = table.reshape(-1, LANES)
    return _graph_embed(tok, span, table4)
